# baseline (speedup 1.0000x reference)
import jax
import jax.numpy as jnp
from jax.experimental import pallas as pl


def _k1_body(x1m2_ref, x2_ref, f1_ref, f2_ref, w1_ref, b1_ref,
             y_ref, s_ref, q_ref, *, S, NBLK):
    b = pl.program_id(0)
    n = pl.program_id(1)
    x1m2 = x1m2_ref[0]
    x2 = x2_ref[0]
    f2 = f2_ref[0]
    sq1 = 0.25 * jnp.sum(x1m2 * x1m2, axis=1, keepdims=True)
    sq2 = jnp.sum(x2 * x2, axis=1, keepdims=True)
    dist = (sq1 + sq2.T) + jnp.dot(
        x1m2, x2.T, preferred_element_type=jnp.float32)

    work = dist
    sel = jnp.zeros((NBLK, S), jnp.float32)
    for _ in range(3):
        m = jnp.min(work, axis=1, keepdims=True)
        mask = work <= m
        r = 1.0 / (jnp.maximum(m, 0.0) + 1e-8)
        sel = jnp.where(mask, r, sel)
        work = jnp.where(mask, jnp.float32(jnp.inf), work)
    rsum = jnp.sum(sel, axis=1, keepdims=True)
    wmat = sel * (1.0 / rsum)

    interp = jnp.dot(wmat, f2, preferred_element_type=jnp.float32)
    concat = jnp.concatenate([interp, f1_ref[0]], axis=1)
    y = jnp.dot(concat, w1_ref[...],
                preferred_element_type=jnp.float32) + b1_ref[0]
    y_ref[0] = y

    @pl.when((b == 0) & (n == 0))
    def _():
        s_ref[...] = jnp.zeros_like(s_ref)
        q_ref[...] = jnp.zeros_like(q_ref)

    s_ref[0, :] += jnp.sum(y, axis=0)
    q_ref[0, :] += jnp.sum(y * y, axis=0)


def _k2_body(y_ref, sc_ref, sh_ref, w2_ref, b2_ref, o_ref, s_ref, q_ref):
    i = pl.program_id(0)
    z = jnp.maximum(y_ref[...] * sc_ref[0] + sh_ref[0], 0.0)
    y2 = jnp.dot(z, w2_ref[...], preferred_element_type=jnp.float32) \
        + b2_ref[0]
    o_ref[...] = y2

    @pl.when(i == 0)
    def _():
        s_ref[...] = jnp.zeros_like(s_ref)
        q_ref[...] = jnp.zeros_like(q_ref)

    s_ref[0, :] += jnp.sum(y2, axis=0)
    q_ref[0, :] += jnp.sum(y2 * y2, axis=0)


def _k3_body(y_ref, sc_ref, sh_ref, o_ref):
    o_ref[...] = jnp.maximum(y_ref[...] * sc_ref[0] + sh_ref[0], 0.0)


def _affine(ssum, ssq, count, g, be, eps=1e-5):
    mean = ssum[0] / count
    var = ssq[0] / count - mean * mean
    scale = g / jnp.sqrt(var + eps)
    shift = be - mean * scale
    return scale.reshape(1, -1), shift.reshape(1, -1)


def kernel(xyz1, xyz2, features1, features2, W1, b1, g1, be1, W2, b2, g2,
           be2, interpret=False):
    B, N, _ = xyz1.shape
    S = xyz2.shape[1]
    C1 = features1.shape[-1]
    C2 = features2.shape[-1]
    H = W1.shape[0]
    Cin = C1 + C2
    NBLK = 512
    MBLK = 1024

    W1t = W1.T
    W2t = W2.T
    b1r = b1.reshape(1, H)
    b2r = b2.reshape(1, H)

    import functools
    k1 = functools.partial(_k1_body, S=S, NBLK=NBLK)
    y1, s1, q1 = pl.pallas_call(
        k1,
        grid=(B, N // NBLK),
        in_specs=[
            pl.BlockSpec((1, NBLK, 3), lambda b, n: (b, n, 0)),
            pl.BlockSpec((1, S, 3), lambda b, n: (b, 0, 0)),
            pl.BlockSpec((1, NBLK, C1), lambda b, n: (b, n, 0)),
            pl.BlockSpec((1, S, C2), lambda b, n: (b, 0, 0)),
            pl.BlockSpec((Cin, H), lambda b, n: (0, 0)),
            pl.BlockSpec((1, H), lambda b, n: (0, 0)),
        ],
        out_specs=[
            pl.BlockSpec((1, NBLK, H), lambda b, n: (b, n, 0)),
            pl.BlockSpec((1, H), lambda b, n: (0, 0)),
            pl.BlockSpec((1, H), lambda b, n: (0, 0)),
        ],
        out_shape=[
            jax.ShapeDtypeStruct((B, N, H), jnp.float32),
            jax.ShapeDtypeStruct((1, H), jnp.float32),
            jax.ShapeDtypeStruct((1, H), jnp.float32),
        ],
        interpret=interpret,
    )(-2.0 * xyz1, xyz2, features1, features2, W1t, b1r)

    sc1, sh1 = _affine(s1, q1, B * N, g1, be1)

    y1f = y1.reshape(B * N, H)
    y2, s2, q2 = pl.pallas_call(
        _k2_body,
        grid=(B * N // MBLK,),
        in_specs=[
            pl.BlockSpec((MBLK, H), lambda i: (i, 0)),
            pl.BlockSpec((1, H), lambda i: (0, 0)),
            pl.BlockSpec((1, H), lambda i: (0, 0)),
            pl.BlockSpec((H, H), lambda i: (0, 0)),
            pl.BlockSpec((1, H), lambda i: (0, 0)),
        ],
        out_specs=[
            pl.BlockSpec((MBLK, H), lambda i: (i, 0)),
            pl.BlockSpec((1, H), lambda i: (0, 0)),
            pl.BlockSpec((1, H), lambda i: (0, 0)),
        ],
        out_shape=[
            jax.ShapeDtypeStruct((B * N, H), jnp.float32),
            jax.ShapeDtypeStruct((1, H), jnp.float32),
            jax.ShapeDtypeStruct((1, H), jnp.float32),
        ],
        interpret=interpret,
    )(y1f, sc1, sh1, W2t, b2r)

    sc2, sh2 = _affine(s2, q2, B * N, g2, be2)

    out = pl.pallas_call(
        _k3_body,
        grid=(B * N // MBLK,),
        in_specs=[
            pl.BlockSpec((MBLK, H), lambda i: (i, 0)),
            pl.BlockSpec((1, H), lambda i: (0, 0)),
            pl.BlockSpec((1, H), lambda i: (0, 0)),
        ],
        out_specs=pl.BlockSpec((MBLK, H), lambda i: (i, 0)),
        out_shape=jax.ShapeDtypeStruct((B * N, H), jnp.float32),
        interpret=interpret,
    )(y2, sc2, sh2)

    return out.reshape(B, N, H)


# speedup vs baseline: 1.0753x; 1.0753x over previous
import jax
import jax.numpy as jnp
from jax.experimental import pallas as pl


def _k1_body(x1m2_ref, x2_ref, f1_ref, f2_ref, w1a_ref, w1b_ref, b1_ref,
             y_ref, s_ref, q_ref, *, S, NBLK):
    b = pl.program_id(0)
    n = pl.program_id(1)
    x1m2 = x1m2_ref[0]
    x2 = x2_ref[0]
    f2 = f2_ref[0]
    sq1 = 0.25 * jnp.sum(x1m2 * x1m2, axis=1, keepdims=True)
    sq2 = jnp.sum(x2 * x2, axis=1, keepdims=True)
    dist = (sq1 + sq2.T) + jnp.dot(
        x1m2, x2.T, preferred_element_type=jnp.float32)

    work = dist
    sel = jnp.zeros((NBLK, S), jnp.float32)
    for _ in range(3):
        m = jnp.min(work, axis=1, keepdims=True)
        mask = work <= m
        r = 1.0 / (jnp.maximum(m, 0.0) + 1e-8)
        sel = jnp.where(mask, r, sel)
        work = jnp.where(mask, jnp.float32(jnp.inf), work)
    rsum = jnp.sum(sel, axis=1, keepdims=True)
    wmat = sel * (1.0 / rsum)

    interp = jnp.dot(wmat, f2, preferred_element_type=jnp.float32)
    y = (jnp.dot(interp, w1a_ref[...], preferred_element_type=jnp.float32)
         + jnp.dot(f1_ref[0], w1b_ref[...],
                   preferred_element_type=jnp.float32)
         + b1_ref[0])
    y_ref[0] = y.astype(jnp.bfloat16)

    @pl.when((b == 0) & (n == 0))
    def _():
        s_ref[...] = jnp.zeros_like(s_ref)
        q_ref[...] = jnp.zeros_like(q_ref)

    s_ref[0, :] += jnp.sum(y, axis=0)
    q_ref[0, :] += jnp.sum(y * y, axis=0)


def _k2_body(y_ref, sc_ref, sh_ref, w2_ref, b2_ref, o_ref, s_ref, q_ref):
    i = pl.program_id(0)
    y1 = y_ref[...].astype(jnp.float32)
    z = jnp.maximum(y1 * sc_ref[0] + sh_ref[0], 0.0)
    y2 = jnp.dot(z, w2_ref[...], preferred_element_type=jnp.float32) \
        + b2_ref[0]
    o_ref[...] = y2.astype(jnp.bfloat16)

    @pl.when(i == 0)
    def _():
        s_ref[...] = jnp.zeros_like(s_ref)
        q_ref[...] = jnp.zeros_like(q_ref)

    s_ref[0, :] += jnp.sum(y2, axis=0)
    q_ref[0, :] += jnp.sum(y2 * y2, axis=0)


def _k3_body(y_ref, sc_ref, sh_ref, o_ref):
    y2 = y_ref[...].astype(jnp.float32)
    o_ref[...] = jnp.maximum(y2 * sc_ref[0] + sh_ref[0], 0.0)


def _affine(ssum, ssq, count, g, be, eps=1e-5):
    mean = ssum[0] / count
    var = ssq[0] / count - mean * mean
    scale = g / jnp.sqrt(var + eps)
    shift = be - mean * scale
    return scale.reshape(1, -1), shift.reshape(1, -1)


def kernel(xyz1, xyz2, features1, features2, W1, b1, g1, be1, W2, b2, g2,
           be2, interpret=False):
    B, N, _ = xyz1.shape
    S = xyz2.shape[1]
    C1 = features1.shape[-1]
    C2 = features2.shape[-1]
    H = W1.shape[0]
    Cin = C1 + C2
    NBLK = 512
    MBLK = 1024

    W1t = W1.T
    W2t = W2.T
    b1r = b1.reshape(1, H)
    b2r = b2.reshape(1, H)

    import functools
    k1 = functools.partial(_k1_body, S=S, NBLK=NBLK)
    y1, s1, q1 = pl.pallas_call(
        k1,
        grid=(B, N // NBLK),
        in_specs=[
            pl.BlockSpec((1, NBLK, 3), lambda b, n: (b, n, 0)),
            pl.BlockSpec((1, S, 3), lambda b, n: (b, 0, 0)),
            pl.BlockSpec((1, NBLK, C1), lambda b, n: (b, n, 0)),
            pl.BlockSpec((1, S, C2), lambda b, n: (b, 0, 0)),
            pl.BlockSpec((C2, H), lambda b, n: (0, 0)),
            pl.BlockSpec((C1, H), lambda b, n: (0, 0)),
            pl.BlockSpec((1, H), lambda b, n: (0, 0)),
        ],
        out_specs=[
            pl.BlockSpec((1, NBLK, H), lambda b, n: (b, n, 0)),
            pl.BlockSpec((1, H), lambda b, n: (0, 0)),
            pl.BlockSpec((1, H), lambda b, n: (0, 0)),
        ],
        out_shape=[
            jax.ShapeDtypeStruct((B, N, H), jnp.bfloat16),
            jax.ShapeDtypeStruct((1, H), jnp.float32),
            jax.ShapeDtypeStruct((1, H), jnp.float32),
        ],
        interpret=interpret,
    )(-2.0 * xyz1, xyz2, features1, features2,
      W1t[:C2], W1t[C2:], b1r)

    sc1, sh1 = _affine(s1, q1, B * N, g1, be1)

    y1f = y1.reshape(B * N, H)
    y2, s2, q2 = pl.pallas_call(
        _k2_body,
        grid=(B * N // MBLK,),
        in_specs=[
            pl.BlockSpec((MBLK, H), lambda i: (i, 0)),
            pl.BlockSpec((1, H), lambda i: (0, 0)),
            pl.BlockSpec((1, H), lambda i: (0, 0)),
            pl.BlockSpec((H, H), lambda i: (0, 0)),
            pl.BlockSpec((1, H), lambda i: (0, 0)),
        ],
        out_specs=[
            pl.BlockSpec((MBLK, H), lambda i: (i, 0)),
            pl.BlockSpec((1, H), lambda i: (0, 0)),
            pl.BlockSpec((1, H), lambda i: (0, 0)),
        ],
        out_shape=[
            jax.ShapeDtypeStruct((B * N, H), jnp.bfloat16),
            jax.ShapeDtypeStruct((1, H), jnp.float32),
            jax.ShapeDtypeStruct((1, H), jnp.float32),
        ],
        interpret=interpret,
    )(y1f, sc1, sh1, W2t, b2r)

    sc2, sh2 = _affine(s2, q2, B * N, g2, be2)

    out = pl.pallas_call(
        _k3_body,
        grid=(B * N // MBLK,),
        in_specs=[
            pl.BlockSpec((MBLK, H), lambda i: (i, 0)),
            pl.BlockSpec((1, H), lambda i: (0, 0)),
            pl.BlockSpec((1, H), lambda i: (0, 0)),
        ],
        out_specs=pl.BlockSpec((MBLK, H), lambda i: (i, 0)),
        out_shape=jax.ShapeDtypeStruct((B * N, H), jnp.float32),
        interpret=interpret,
    )(y2, sc2, sh2)

    return out.reshape(B, N, H)


# speedup vs baseline: 1.3021x; 1.2109x over previous
import jax
import jax.numpy as jnp
from jax.experimental import pallas as pl


def _k1_body(x1m2_ref, x2_ref, f1_ref, f2_ref, w1a_ref, w1b_ref, b1_ref,
             y_ref, s_ref, q_ref, *, S, NBLK):
    b = pl.program_id(0)
    n = pl.program_id(1)
    x1m2 = x1m2_ref[0]
    x2 = x2_ref[0]
    f2 = f2_ref[0]
    sq1 = 0.25 * jnp.sum(x1m2 * x1m2, axis=1, keepdims=True)
    sq2 = jnp.sum(x2 * x2, axis=1, keepdims=True)
    dist = (sq1 + sq2.T) + jnp.dot(
        x1m2, x2.T, preferred_element_type=jnp.float32)

    work = dist
    sel = jnp.zeros((NBLK, S), jnp.float32)
    for _ in range(3):
        m = jnp.min(work, axis=1, keepdims=True)
        mask = work <= m
        r = 1.0 / (jnp.maximum(m, 0.0) + 1e-8)
        sel = jnp.where(mask, r, sel)
        work = jnp.where(mask, jnp.float32(jnp.inf), work)
    rsum = jnp.sum(sel, axis=1, keepdims=True)
    wmat = sel * (1.0 / rsum)

    interp = jnp.dot(wmat, f2, preferred_element_type=jnp.float32)
    y = (jnp.dot(interp, w1a_ref[...], preferred_element_type=jnp.float32)
         + jnp.dot(f1_ref[0], w1b_ref[...],
                   preferred_element_type=jnp.float32)
         + b1_ref[0])
    y_ref[0] = y.astype(jnp.bfloat16)

    @pl.when((b == 0) & (n == 0))
    def _():
        s_ref[...] = jnp.zeros_like(s_ref)
        q_ref[...] = jnp.zeros_like(q_ref)

    s_ref[0, :] += jnp.sum(y, axis=0)
    q_ref[0, :] += jnp.sum(y * y, axis=0)


def _k2_body(y_ref, sc_ref, sh_ref, w2_ref, b2_ref, o_ref, s_ref, q_ref):
    i = pl.program_id(0)
    y1 = y_ref[...].astype(jnp.float32)
    z = jnp.maximum(y1 * sc_ref[0] + sh_ref[0], 0.0)
    y2 = jnp.dot(z, w2_ref[...], preferred_element_type=jnp.float32) \
        + b2_ref[0]
    o_ref[...] = y2.astype(jnp.bfloat16)

    @pl.when(i == 0)
    def _():
        s_ref[...] = jnp.zeros_like(s_ref)
        q_ref[...] = jnp.zeros_like(q_ref)

    s_ref[0, :] += jnp.sum(y2, axis=0)
    q_ref[0, :] += jnp.sum(y2 * y2, axis=0)


def _k3_body(y_ref, sc_ref, sh_ref, o_ref):
    y2 = y_ref[...].astype(jnp.float32)
    o_ref[...] = jnp.maximum(y2 * sc_ref[0] + sh_ref[0], 0.0)


def _affine(ssum, ssq, count, g, be, eps=1e-5):
    mean = ssum[0] / count
    var = ssq[0] / count - mean * mean
    scale = g / jnp.sqrt(var + eps)
    shift = be - mean * scale
    return scale.reshape(1, -1), shift.reshape(1, -1)


def kernel(xyz1, xyz2, features1, features2, W1, b1, g1, be1, W2, b2, g2,
           be2, interpret=False):
    B, N, _ = xyz1.shape
    S = xyz2.shape[1]
    C1 = features1.shape[-1]
    C2 = features2.shape[-1]
    H = W1.shape[0]
    Cin = C1 + C2
    NBLK = 1024
    MBLK = 2048

    W1t = W1.T
    W2t = W2.T
    b1r = b1.reshape(1, H)
    b2r = b2.reshape(1, H)

    import functools
    k1 = functools.partial(_k1_body, S=S, NBLK=NBLK)
    y1, s1, q1 = pl.pallas_call(
        k1,
        grid=(B, N // NBLK),
        in_specs=[
            pl.BlockSpec((1, NBLK, 3), lambda b, n: (b, n, 0)),
            pl.BlockSpec((1, S, 3), lambda b, n: (b, 0, 0)),
            pl.BlockSpec((1, NBLK, C1), lambda b, n: (b, n, 0)),
            pl.BlockSpec((1, S, C2), lambda b, n: (b, 0, 0)),
            pl.BlockSpec((C2, H), lambda b, n: (0, 0)),
            pl.BlockSpec((C1, H), lambda b, n: (0, 0)),
            pl.BlockSpec((1, H), lambda b, n: (0, 0)),
        ],
        out_specs=[
            pl.BlockSpec((1, NBLK, H), lambda b, n: (b, n, 0)),
            pl.BlockSpec((1, H), lambda b, n: (0, 0)),
            pl.BlockSpec((1, H), lambda b, n: (0, 0)),
        ],
        out_shape=[
            jax.ShapeDtypeStruct((B, N, H), jnp.bfloat16),
            jax.ShapeDtypeStruct((1, H), jnp.float32),
            jax.ShapeDtypeStruct((1, H), jnp.float32),
        ],
        interpret=interpret,
    )(-2.0 * xyz1, xyz2, features1, features2,
      W1t[:C2], W1t[C2:], b1r)

    sc1, sh1 = _affine(s1, q1, B * N, g1, be1)

    y1f = y1.reshape(B * N, H)
    y2, s2, q2 = pl.pallas_call(
        _k2_body,
        grid=(B * N // MBLK,),
        in_specs=[
            pl.BlockSpec((MBLK, H), lambda i: (i, 0)),
            pl.BlockSpec((1, H), lambda i: (0, 0)),
            pl.BlockSpec((1, H), lambda i: (0, 0)),
            pl.BlockSpec((H, H), lambda i: (0, 0)),
            pl.BlockSpec((1, H), lambda i: (0, 0)),
        ],
        out_specs=[
            pl.BlockSpec((MBLK, H), lambda i: (i, 0)),
            pl.BlockSpec((1, H), lambda i: (0, 0)),
            pl.BlockSpec((1, H), lambda i: (0, 0)),
        ],
        out_shape=[
            jax.ShapeDtypeStruct((B * N, H), jnp.bfloat16),
            jax.ShapeDtypeStruct((1, H), jnp.float32),
            jax.ShapeDtypeStruct((1, H), jnp.float32),
        ],
        interpret=interpret,
    )(y1f, sc1, sh1, W2t, b2r)

    sc2, sh2 = _affine(s2, q2, B * N, g2, be2)

    out = pl.pallas_call(
        _k3_body,
        grid=(B * N // MBLK,),
        in_specs=[
            pl.BlockSpec((MBLK, H), lambda i: (i, 0)),
            pl.BlockSpec((1, H), lambda i: (0, 0)),
            pl.BlockSpec((1, H), lambda i: (0, 0)),
        ],
        out_specs=pl.BlockSpec((MBLK, H), lambda i: (i, 0)),
        out_shape=jax.ShapeDtypeStruct((B * N, H), jnp.float32),
        interpret=interpret,
    )(y2, sc2, sh2)

    return out.reshape(B, N, H)


# speedup vs baseline: 1.4309x; 1.0989x over previous
import jax
import jax.numpy as jnp
from jax.experimental import pallas as pl


def _k1_body(x1m2_ref, x2_ref, f1_ref, f2_ref, w1a_ref, w1b_ref, b1_ref,
             y_ref, s_ref, q_ref, *, S, NBLK):
    b = pl.program_id(0)
    n = pl.program_id(1)
    x1m2 = x1m2_ref[0]
    x2 = x2_ref[0]
    f2 = f2_ref[0]
    sq1 = 0.25 * jnp.sum(x1m2 * x1m2, axis=1, keepdims=True)
    sq2 = jnp.sum(x2 * x2, axis=1, keepdims=True)
    dist = (sq1 + sq2.T) + jnp.dot(
        x1m2, x2.T, preferred_element_type=jnp.float32)

    work = dist
    sel = jnp.zeros((NBLK, S), jnp.float32)
    for _ in range(3):
        m = jnp.min(work, axis=1, keepdims=True)
        mask = work <= m
        r = 1.0 / (jnp.maximum(m, 0.0) + 1e-8)
        sel = jnp.where(mask, r, sel)
        work = jnp.where(mask, jnp.float32(jnp.inf), work)
    rsum = jnp.sum(sel, axis=1, keepdims=True)
    wmat = sel * (1.0 / rsum)

    interp = jnp.dot(wmat, f2, preferred_element_type=jnp.float32)
    y = (jnp.dot(interp, w1a_ref[...], preferred_element_type=jnp.float32)
         + jnp.dot(f1_ref[0], w1b_ref[...],
                   preferred_element_type=jnp.float32)
         + b1_ref[0])
    y_ref[0] = y.astype(jnp.bfloat16)

    @pl.when((b == 0) & (n == 0))
    def _():
        s_ref[...] = jnp.zeros_like(s_ref)
        q_ref[...] = jnp.zeros_like(q_ref)

    s_ref[0, :] += jnp.sum(y, axis=0)
    q_ref[0, :] += jnp.sum(y * y, axis=0)


def _k2_body(y_ref, sc_ref, sh_ref, w2_ref, b2_ref, o_ref, s_ref, q_ref):
    i = pl.program_id(0)
    y1 = y_ref[...].astype(jnp.float32)
    z = jnp.maximum(y1 * sc_ref[0] + sh_ref[0], 0.0)
    y2 = jnp.dot(z, w2_ref[...], preferred_element_type=jnp.float32) \
        + b2_ref[0]
    o_ref[...] = y2.astype(jnp.bfloat16)

    @pl.when(i == 0)
    def _():
        s_ref[...] = jnp.zeros_like(s_ref)
        q_ref[...] = jnp.zeros_like(q_ref)

    s_ref[0, :] += jnp.sum(y2, axis=0)
    q_ref[0, :] += jnp.sum(y2 * y2, axis=0)


def _k3_body(y_ref, sc_ref, sh_ref, o_ref):
    y2 = y_ref[...].astype(jnp.float32)
    o_ref[...] = jnp.maximum(y2 * sc_ref[0] + sh_ref[0], 0.0)


def _affine(ssum, ssq, count, g, be, eps=1e-5):
    mean = ssum[0] / count
    var = ssq[0] / count - mean * mean
    scale = g / jnp.sqrt(var + eps)
    shift = be - mean * scale
    return scale.reshape(1, -1), shift.reshape(1, -1)


def kernel(xyz1, xyz2, features1, features2, W1, b1, g1, be1, W2, b2, g2,
           be2, interpret=False):
    B, N, _ = xyz1.shape
    S = xyz2.shape[1]
    C1 = features1.shape[-1]
    C2 = features2.shape[-1]
    H = W1.shape[0]
    Cin = C1 + C2
    NBLK = 2048
    MBLK = 4096

    W1t = W1.T
    W2t = W2.T
    b1r = b1.reshape(1, H)
    b2r = b2.reshape(1, H)

    import functools
    k1 = functools.partial(_k1_body, S=S, NBLK=NBLK)
    y1, s1, q1 = pl.pallas_call(
        k1,
        grid=(B, N // NBLK),
        in_specs=[
            pl.BlockSpec((1, NBLK, 3), lambda b, n: (b, n, 0)),
            pl.BlockSpec((1, S, 3), lambda b, n: (b, 0, 0)),
            pl.BlockSpec((1, NBLK, C1), lambda b, n: (b, n, 0)),
            pl.BlockSpec((1, S, C2), lambda b, n: (b, 0, 0)),
            pl.BlockSpec((C2, H), lambda b, n: (0, 0)),
            pl.BlockSpec((C1, H), lambda b, n: (0, 0)),
            pl.BlockSpec((1, H), lambda b, n: (0, 0)),
        ],
        out_specs=[
            pl.BlockSpec((1, NBLK, H), lambda b, n: (b, n, 0)),
            pl.BlockSpec((1, H), lambda b, n: (0, 0)),
            pl.BlockSpec((1, H), lambda b, n: (0, 0)),
        ],
        out_shape=[
            jax.ShapeDtypeStruct((B, N, H), jnp.bfloat16),
            jax.ShapeDtypeStruct((1, H), jnp.float32),
            jax.ShapeDtypeStruct((1, H), jnp.float32),
        ],
        interpret=interpret,
    )(-2.0 * xyz1, xyz2, features1, features2,
      W1t[:C2], W1t[C2:], b1r)

    sc1, sh1 = _affine(s1, q1, B * N, g1, be1)

    y1f = y1.reshape(B * N, H)
    y2, s2, q2 = pl.pallas_call(
        _k2_body,
        grid=(B * N // MBLK,),
        in_specs=[
            pl.BlockSpec((MBLK, H), lambda i: (i, 0)),
            pl.BlockSpec((1, H), lambda i: (0, 0)),
            pl.BlockSpec((1, H), lambda i: (0, 0)),
            pl.BlockSpec((H, H), lambda i: (0, 0)),
            pl.BlockSpec((1, H), lambda i: (0, 0)),
        ],
        out_specs=[
            pl.BlockSpec((MBLK, H), lambda i: (i, 0)),
            pl.BlockSpec((1, H), lambda i: (0, 0)),
            pl.BlockSpec((1, H), lambda i: (0, 0)),
        ],
        out_shape=[
            jax.ShapeDtypeStruct((B * N, H), jnp.bfloat16),
            jax.ShapeDtypeStruct((1, H), jnp.float32),
            jax.ShapeDtypeStruct((1, H), jnp.float32),
        ],
        interpret=interpret,
    )(y1f, sc1, sh1, W2t, b2r)

    sc2, sh2 = _affine(s2, q2, B * N, g2, be2)

    out = pl.pallas_call(
        _k3_body,
        grid=(B * N // MBLK,),
        in_specs=[
            pl.BlockSpec((MBLK, H), lambda i: (i, 0)),
            pl.BlockSpec((1, H), lambda i: (0, 0)),
            pl.BlockSpec((1, H), lambda i: (0, 0)),
        ],
        out_specs=pl.BlockSpec((MBLK, H), lambda i: (i, 0)),
        out_shape=jax.ShapeDtypeStruct((B * N, H), jnp.float32),
        interpret=interpret,
    )(y2, sc2, sh2)

    return out.reshape(B, N, H)


# speedup vs baseline: 1.4956x; 1.0453x over previous
import jax
import jax.numpy as jnp
from jax.experimental import pallas as pl


def _k1_body(x1m2_ref, x2_ref, f1_ref, f2_ref, w1a_ref, w1b_ref, b1_ref,
             y_ref, s_ref, q_ref, *, S, NBLK):
    b = pl.program_id(0)
    n = pl.program_id(1)
    x1m2 = x1m2_ref[0]
    x2 = x2_ref[0]
    f2 = f2_ref[0]
    sq1 = 0.25 * jnp.sum(x1m2 * x1m2, axis=1, keepdims=True)
    sq2 = jnp.sum(x2 * x2, axis=1, keepdims=True)
    dist = (sq1 + sq2.T) + jnp.dot(
        x1m2, x2.T, preferred_element_type=jnp.float32)

    work = dist
    sel = jnp.zeros((NBLK, S), jnp.float32)
    for _ in range(3):
        m = jnp.min(work, axis=1, keepdims=True)
        mask = work <= m
        r = 1.0 / (jnp.maximum(m, 0.0) + 1e-8)
        sel = jnp.where(mask, r, sel)
        work = jnp.where(mask, jnp.float32(jnp.inf), work)
    rsum = jnp.sum(sel, axis=1, keepdims=True)
    wmat = sel * (1.0 / rsum)

    interp = jnp.dot(wmat, f2, preferred_element_type=jnp.float32)
    y = (jnp.dot(interp, w1a_ref[...], preferred_element_type=jnp.float32)
         + jnp.dot(f1_ref[0], w1b_ref[...],
                   preferred_element_type=jnp.float32)
         + b1_ref[0])
    y_ref[0] = y.astype(jnp.bfloat16)

    @pl.when((b == 0) & (n == 0))
    def _():
        s_ref[...] = jnp.zeros_like(s_ref)
        q_ref[...] = jnp.zeros_like(q_ref)

    s_ref[0, :] += jnp.sum(y, axis=0)
    q_ref[0, :] += jnp.sum(y * y, axis=0)


def _k2_body(y_ref, sc_ref, sh_ref, w2_ref, b2_ref, o_ref, s_ref, q_ref):
    i = pl.program_id(0)
    y1 = y_ref[...].astype(jnp.float32)
    z = jnp.maximum(y1 * sc_ref[0] + sh_ref[0], 0.0)
    y2 = jnp.dot(z, w2_ref[...], preferred_element_type=jnp.float32) \
        + b2_ref[0]
    o_ref[...] = y2.astype(jnp.bfloat16)

    @pl.when(i == 0)
    def _():
        s_ref[...] = jnp.zeros_like(s_ref)
        q_ref[...] = jnp.zeros_like(q_ref)

    s_ref[0, :] += jnp.sum(y2, axis=0)
    q_ref[0, :] += jnp.sum(y2 * y2, axis=0)


def _k3_body(y_ref, sc_ref, sh_ref, o_ref):
    y2 = y_ref[...].astype(jnp.float32)
    o_ref[...] = jnp.maximum(y2 * sc_ref[0] + sh_ref[0], 0.0)


def _affine(ssum, ssq, count, g, be, eps=1e-5):
    mean = ssum[0] / count
    var = ssq[0] / count - mean * mean
    scale = g / jnp.sqrt(var + eps)
    shift = be - mean * scale
    return scale.reshape(1, -1), shift.reshape(1, -1)


def kernel(xyz1, xyz2, features1, features2, W1, b1, g1, be1, W2, b2, g2,
           be2, interpret=False):
    B, N, _ = xyz1.shape
    S = xyz2.shape[1]
    C1 = features1.shape[-1]
    C2 = features2.shape[-1]
    H = W1.shape[0]
    Cin = C1 + C2
    NBLK = 4096
    MBLK = 8192

    W1t = W1.T
    W2t = W2.T
    b1r = b1.reshape(1, H)
    b2r = b2.reshape(1, H)

    import functools
    k1 = functools.partial(_k1_body, S=S, NBLK=NBLK)
    y1, s1, q1 = pl.pallas_call(
        k1,
        grid=(B, N // NBLK),
        in_specs=[
            pl.BlockSpec((1, NBLK, 3), lambda b, n: (b, n, 0)),
            pl.BlockSpec((1, S, 3), lambda b, n: (b, 0, 0)),
            pl.BlockSpec((1, NBLK, C1), lambda b, n: (b, n, 0)),
            pl.BlockSpec((1, S, C2), lambda b, n: (b, 0, 0)),
            pl.BlockSpec((C2, H), lambda b, n: (0, 0)),
            pl.BlockSpec((C1, H), lambda b, n: (0, 0)),
            pl.BlockSpec((1, H), lambda b, n: (0, 0)),
        ],
        out_specs=[
            pl.BlockSpec((1, NBLK, H), lambda b, n: (b, n, 0)),
            pl.BlockSpec((1, H), lambda b, n: (0, 0)),
            pl.BlockSpec((1, H), lambda b, n: (0, 0)),
        ],
        out_shape=[
            jax.ShapeDtypeStruct((B, N, H), jnp.bfloat16),
            jax.ShapeDtypeStruct((1, H), jnp.float32),
            jax.ShapeDtypeStruct((1, H), jnp.float32),
        ],
        interpret=interpret,
    )(-2.0 * xyz1, xyz2, features1, features2,
      W1t[:C2], W1t[C2:], b1r)

    sc1, sh1 = _affine(s1, q1, B * N, g1, be1)

    y1f = y1.reshape(B * N, H)
    y2, s2, q2 = pl.pallas_call(
        _k2_body,
        grid=(B * N // MBLK,),
        in_specs=[
            pl.BlockSpec((MBLK, H), lambda i: (i, 0)),
            pl.BlockSpec((1, H), lambda i: (0, 0)),
            pl.BlockSpec((1, H), lambda i: (0, 0)),
            pl.BlockSpec((H, H), lambda i: (0, 0)),
            pl.BlockSpec((1, H), lambda i: (0, 0)),
        ],
        out_specs=[
            pl.BlockSpec((MBLK, H), lambda i: (i, 0)),
            pl.BlockSpec((1, H), lambda i: (0, 0)),
            pl.BlockSpec((1, H), lambda i: (0, 0)),
        ],
        out_shape=[
            jax.ShapeDtypeStruct((B * N, H), jnp.bfloat16),
            jax.ShapeDtypeStruct((1, H), jnp.float32),
            jax.ShapeDtypeStruct((1, H), jnp.float32),
        ],
        interpret=interpret,
    )(y1f, sc1, sh1, W2t, b2r)

    sc2, sh2 = _affine(s2, q2, B * N, g2, be2)

    out = pl.pallas_call(
        _k3_body,
        grid=(B * N // MBLK,),
        in_specs=[
            pl.BlockSpec((MBLK, H), lambda i: (i, 0)),
            pl.BlockSpec((1, H), lambda i: (0, 0)),
            pl.BlockSpec((1, H), lambda i: (0, 0)),
        ],
        out_specs=pl.BlockSpec((MBLK, H), lambda i: (i, 0)),
        out_shape=jax.ShapeDtypeStruct((B * N, H), jnp.float32),
        interpret=interpret,
    )(y2, sc2, sh2)

    return out.reshape(B, N, H)
